# baseline (speedup 1.0000x reference)
import functools

import jax
import jax.numpy as jnp
from jax import lax
from jax.experimental import pallas as pl
from jax.experimental.pallas import tpu as pltpu
from jax.experimental.pallas import tpu_sc as plsc

N = 10000
NN = 16
F = 128
EF = 16

_NC = 2
_NS = 16
_NW = _NC * _NS


_BN = 400
_GRID = N // _BN


def _edge_fc_body(x_ref, w0_ref, b0_ref, w1_ref, b1_ref, o_ref):
    x = x_ref[...]
    h = jnp.maximum(x @ w0_ref[...] + b0_ref[...], 0.0)
    o_ref[...] = jnp.tanh(h @ w1_ref[...] + b1_ref[...])


def _edge_fc(x2, w0b, b0b, w1b, b1b):
    rows = x2.shape[0]
    blk = rows // 4
    return pl.pallas_call(
        _edge_fc_body,
        grid=(4,),
        in_specs=[
            pl.BlockSpec((blk, 128), lambda i: (i, 0)),
            pl.BlockSpec((128, 128), lambda i: (0, 0)),
            pl.BlockSpec((1, 128), lambda i: (0, 0)),
            pl.BlockSpec((128, 128), lambda i: (0, 0)),
            pl.BlockSpec((1, 128), lambda i: (0, 0)),
        ],
        out_specs=pl.BlockSpec((blk, 128), lambda i: (i, 0)),
        out_shape=jax.ShapeDtypeStruct((rows, 128), jnp.float32),
    )(x2, w0b, b0b, w1b, b1b)


_GCH = 128
_NCHUNKS = (N * NN) // _GCH
_MAXIT = (_NCHUNKS + _NW - 1) // _NW
assert _MAXIT % 2 == 0


def _gather_body(table_hbm, idx_hbm, out_hbm, spmem, idx0, idx1,
                 buf0, buf1, sem0, sem1, sem_s):
    c = lax.axis_index("c")
    s = lax.axis_index("s")
    wid = s * _NC + c

    @pl.when(s == 0)
    def _stage():
        pltpu.async_copy(table_hbm, spmem, sem_s).wait()

    plsc.subcore_barrier()

    def _issue(ch, idx_v, buf, sem):
        pltpu.sync_copy(idx_hbm.at[ch], idx_v)
        pltpu.async_copy(spmem.at[idx_v], buf, sem)

    def _wait(buf, sem):
        pltpu.make_async_copy(spmem.at[idx0], buf, sem).wait()

    def _scat(ch, buf):
        pltpu.sync_copy(buf, out_hbm.at[pl.ds(ch * _GCH, _GCH)])

    _issue(wid, idx0, buf0, sem0)

    def body(k, carry):
        c0 = wid + _NW * 2 * k
        c1 = c0 + _NW
        c2 = c1 + _NW
        _wait(buf0, sem0)

        @pl.when(c1 < _NCHUNKS)
        def _():
            _issue(c1, idx1, buf1, sem1)

        _scat(c0, buf0)

        @pl.when(c2 < _NCHUNKS)
        def _():
            _issue(c2, idx0, buf0, sem0)

        @pl.when(c1 < _NCHUNKS)
        def _():
            _wait(buf1, sem1)
            _scat(c1, buf1)

        return carry

    lax.fori_loop(0, _MAXIT // 2, body, 0)


_gather = functools.partial(
    pl.kernel,
    out_type=jax.ShapeDtypeStruct((N * NN, F), jnp.float32),
    mesh=plsc.VectorSubcoreMesh(core_axis_name="c", subcore_axis_name="s"),
    scratch_types=[
        pltpu.VMEM_SHARED((N, F), jnp.float32),
        pltpu.VMEM((_GCH,), jnp.int32),
        pltpu.VMEM((_GCH,), jnp.int32),
        pltpu.VMEM((_GCH, F), jnp.float32),
        pltpu.VMEM((_GCH, F), jnp.float32),
        pltpu.SemaphoreType.DMA,
        pltpu.SemaphoreType.DMA,
        pltpu.SemaphoreType.DMA,
    ],
)(_gather_body)


def _mp_core(ng_ref, e_ref, res_ref, w2_ref, t_ref):
    B = res_ref.shape[0]
    G = B // 16
    bf16 = jnp.bfloat16
    ep = e_ref[...]
    e = jnp.concatenate([ep[:, a * EF:(a + 1) * EF] for a in range(8)],
                        axis=0).astype(bf16)
    ng = ng_ref[...].astype(bf16)
    ci = lax.broadcasted_iota(jnp.int32, (16, 256), 1)
    ri = lax.broadcasted_iota(jnp.int32, (16, 256), 0)
    spread = (ci // 16 == ri).astype(bf16)
    r2 = lax.broadcasted_iota(jnp.int32, (256, 256), 0)
    c2 = lax.broadcasted_iota(jnp.int32, (256, 256), 1)
    mask = ((r2 // 16) == (c2 % 16)).astype(bf16)
    for g in range(G):
        eg = e[g * 256:(g + 1) * 256, :]
        ngg = ng[g * 256:(g + 1) * 256, :]
        erep = lax.dot_general(eg, spread, (((1,), (0,)), ((), ())),
                               preferred_element_type=jnp.float32)
        ablk = erep.astype(bf16) * mask
        rg = lax.dot_general(ablk, ngg, (((0,), (0,)), ((), ())),
                             preferred_element_type=jnp.float32)
        for n in range(16):
            t_ref[n, g * 16:(g + 1) * 16, :] = rg[n * 16:(n + 1) * 16, :].astype(bf16)
    acc = jnp.zeros((B, 128), jnp.float32)
    for n in range(16):
        acc = acc + lax.dot_general(t_ref[n], w2_ref[n],
                                    (((1,), (0,)), ((), ())),
                                    preferred_element_type=jnp.float32)
    return jnp.maximum(acc * (1.0 / NN), 0.0) + res_ref[...]


def _mp_body(ng_ref, e_ref, res_ref, w2_ref, o_ref, t_ref):
    o_ref[...] = _mp_core(ng_ref, e_ref, res_ref, w2_ref, t_ref)


def _mp_fc_body(ng_ref, e_ref, res_ref, w2_ref, fw0_ref, fb0_ref, fw1_ref,
                fb1_ref, o_ref, t_ref):
    nodes2 = _mp_core(ng_ref, e_ref, res_ref, w2_ref, t_ref)
    h = jnp.maximum(nodes2 @ fw0_ref[...] + fb0_ref[...], 0.0)
    o_ref[...] = jnp.tanh(h @ fw1_ref[...] + fb1_ref[...])


_MP_SPECS = [
    pl.BlockSpec((_BN * NN, 128), lambda i: (i, 0)),
    pl.BlockSpec((_BN * NN * EF // 128, 128), lambda i: (i, 0)),
    pl.BlockSpec((_BN, 128), lambda i: (i, 0)),
    pl.BlockSpec((16, 128, 128), lambda i: (0, 0, 0)),
]
_FC_SPECS = [
    pl.BlockSpec((128, 128), lambda i: (0, 0)),
    pl.BlockSpec((1, 128), lambda i: (0, 0)),
    pl.BlockSpec((128, 128), lambda i: (0, 0)),
    pl.BlockSpec((1, 128), lambda i: (0, 0)),
]


def _mp(ng, e, res, w2):
    return pl.pallas_call(
        _mp_body,
        grid=(_GRID,),
        in_specs=_MP_SPECS,
        out_specs=pl.BlockSpec((_BN, 128), lambda i: (i, 0)),
        out_shape=jax.ShapeDtypeStruct((N, 128), jnp.float32),
        scratch_shapes=[pltpu.VMEM((16, _BN, 128), jnp.bfloat16)],
    )(ng, e, res, w2)


def _mp_fc(ng, e, res, w2, fw0, fb0, fw1, fb1):
    return pl.pallas_call(
        _mp_fc_body,
        grid=(_GRID,),
        in_specs=_MP_SPECS + _FC_SPECS,
        out_specs=pl.BlockSpec((_BN, 128), lambda i: (i, 0)),
        out_shape=jax.ShapeDtypeStruct((N, 128), jnp.float32),
        scratch_shapes=[pltpu.VMEM((16, _BN, 128), jnp.bfloat16)],
    )(ng, e, res, w2, fw0, fb0, fw1, fb1)


def kernel(node_input, nlist_input, edge_input, edge_w0, edge_b0, edge_w1,
           edge_b1, mp_w0, mp_w1, fc_w0, fc_b0, fc_w1, fc_b1):
    f32 = jnp.float32
    eye8 = jnp.eye(8, dtype=f32)
    w0b = jnp.kron(eye8, edge_w0.astype(f32))
    w1b = jnp.kron(eye8, edge_w1.astype(f32))
    b0b = jnp.tile(edge_b0.astype(f32), 8).reshape(1, 128)
    b1b = jnp.tile(edge_b1.astype(f32), 8).reshape(1, 128)

    x2 = (edge_input.astype(f32)
          .reshape(_GRID, 8, _BN * NN // 8, EF)
          .transpose(0, 2, 1, 3)
          .reshape(N * NN * EF // 128, 128))
    edges_packed = _edge_fc(x2, w0b, b0b, w1b, b1b)

    idx = nlist_input.astype(jnp.int32).reshape(_NCHUNKS, _GCH)
    w2a = jnp.transpose(mp_w0.astype(jnp.bfloat16), (1, 0, 2))
    w2b = jnp.transpose(mp_w1.astype(jnp.bfloat16), (1, 0, 2))

    node_f32 = node_input.astype(f32)
    ng1 = _gather(node_f32, idx)
    nodes1 = _mp(ng1, edges_packed, node_f32, w2a)
    ng2 = _gather(nodes1, idx)
    return _mp_fc(ng2, edges_packed, nodes1, w2b,
                  fc_w0.astype(f32), fc_b0.astype(f32).reshape(1, 128),
                  fc_w1.astype(f32), fc_b1.astype(f32).reshape(1, 128))


# speedup vs baseline: 1.0874x; 1.0874x over previous
import functools

import jax
import jax.numpy as jnp
from jax import lax
from jax.experimental import pallas as pl
from jax.experimental.pallas import tpu as pltpu
from jax.experimental.pallas import tpu_sc as plsc

N = 10000
NN = 16
F = 128
EF = 16

_NC = 2
_NS = 16
_NW = _NC * _NS


_BN = 400
_GRID = N // _BN


def _edge_fc_body(x_ref, w0_ref, b0_ref, w1_ref, b1_ref, o_ref):
    x = x_ref[...]
    h = jnp.maximum(x @ w0_ref[...] + b0_ref[...], 0.0)
    o_ref[...] = jnp.tanh(h @ w1_ref[...] + b1_ref[...])


def _edge_fc(x2, w0b, b0b, w1b, b1b):
    rows = x2.shape[0]
    blk = rows // 4
    return pl.pallas_call(
        _edge_fc_body,
        grid=(4,),
        in_specs=[
            pl.BlockSpec((blk, 128), lambda i: (i, 0)),
            pl.BlockSpec((128, 128), lambda i: (0, 0)),
            pl.BlockSpec((1, 128), lambda i: (0, 0)),
            pl.BlockSpec((128, 128), lambda i: (0, 0)),
            pl.BlockSpec((1, 128), lambda i: (0, 0)),
        ],
        out_specs=pl.BlockSpec((blk, 128), lambda i: (i, 0)),
        out_shape=jax.ShapeDtypeStruct((rows, 128), jnp.float32),
    )(x2, w0b, b0b, w1b, b1b)


_RPW = (N * NN) // _NW
_CH = 120
_NFULL = _RPW // _CH
_TAIL = _RPW - _NFULL * _CH
_IDXROWS = _NFULL + 1


def _gather_body(table_hbm, idx_hbm, out_hbm, spmem, idx_v, buf0, buf1,
                 sem0, sem1, sem_s):
    c = lax.axis_index("c")
    s = lax.axis_index("s")
    wid = s * _NC + c

    @pl.when(s == 0)
    def _stage():
        pltpu.async_copy(table_hbm, spmem, sem_s).wait()

    plsc.subcore_barrier()
    pltpu.sync_copy(idx_hbm.at[wid], idx_v)
    base = wid * _RPW

    def _gath(g, buf, sem):
        return pltpu.async_copy(spmem.at[idx_v.at[g]], buf, sem)

    def _scat(g, buf):
        pltpu.sync_copy(buf, out_hbm.at[pl.ds(base + g * _CH, _CH)])

    _gath(0, buf0, sem0)

    def body(it, carry):
        a = 2 * it
        pltpu.make_async_copy(spmem.at[idx_v.at[a]], buf0, sem0).wait()
        _gath(a + 1, buf1, sem1)
        _scat(a, buf0)
        pltpu.make_async_copy(spmem.at[idx_v.at[a + 1]], buf1, sem1).wait()
        _gath(a + 2, buf0, sem0)
        _scat(a + 1, buf1)
        return carry

    lax.fori_loop(0, _NFULL // 2, body, 0)
    last = _NFULL - 1
    pltpu.make_async_copy(spmem.at[idx_v.at[last]], buf0, sem0).wait()
    pltpu.async_copy(spmem.at[idx_v.at[_NFULL, pl.ds(0, _TAIL)]],
                     buf1.at[pl.ds(0, _TAIL)], sem1)
    _scat(last, buf0)
    pltpu.make_async_copy(spmem.at[idx_v.at[_NFULL, pl.ds(0, _TAIL)]],
                          buf1.at[pl.ds(0, _TAIL)], sem1).wait()
    pltpu.sync_copy(buf1.at[pl.ds(0, _TAIL)],
                    out_hbm.at[pl.ds(base + _NFULL * _CH, _TAIL)])


_gather = functools.partial(
    pl.kernel,
    out_type=jax.ShapeDtypeStruct((N * NN, F), jnp.float32),
    mesh=plsc.VectorSubcoreMesh(core_axis_name="c", subcore_axis_name="s"),
    scratch_types=[
        pltpu.VMEM_SHARED((N, F), jnp.float32),
        pltpu.VMEM((_IDXROWS, _CH), jnp.int32),
        pltpu.VMEM((_CH, F), jnp.float32),
        pltpu.VMEM((_CH, F), jnp.float32),
        pltpu.SemaphoreType.DMA,
        pltpu.SemaphoreType.DMA,
        pltpu.SemaphoreType.DMA,
    ],
)(_gather_body)


def _mp_core(ng_ref, e_ref, res_ref, w2_ref, t_ref):
    B = res_ref.shape[0]
    G = B // 16
    bf16 = jnp.bfloat16
    ep = e_ref[...]
    e = jnp.concatenate([ep[:, a * EF:(a + 1) * EF] for a in range(8)],
                        axis=0).astype(bf16)
    ng = ng_ref[...].astype(bf16)
    ci = lax.broadcasted_iota(jnp.int32, (16, 256), 1)
    ri = lax.broadcasted_iota(jnp.int32, (16, 256), 0)
    spread = (ci // 16 == ri).astype(bf16)
    r2 = lax.broadcasted_iota(jnp.int32, (256, 256), 0)
    c2 = lax.broadcasted_iota(jnp.int32, (256, 256), 1)
    mask = ((r2 // 16) == (c2 % 16)).astype(bf16)
    for g in range(G):
        eg = e[g * 256:(g + 1) * 256, :]
        ngg = ng[g * 256:(g + 1) * 256, :]
        erep = lax.dot_general(eg, spread, (((1,), (0,)), ((), ())),
                               preferred_element_type=jnp.float32)
        ablk = erep.astype(bf16) * mask
        rg = lax.dot_general(ablk, ngg, (((0,), (0,)), ((), ())),
                             preferred_element_type=jnp.float32)
        for n in range(16):
            t_ref[n, g * 16:(g + 1) * 16, :] = rg[n * 16:(n + 1) * 16, :].astype(bf16)
    acc = jnp.zeros((B, 128), jnp.float32)
    for n in range(16):
        acc = acc + lax.dot_general(t_ref[n], w2_ref[n],
                                    (((1,), (0,)), ((), ())),
                                    preferred_element_type=jnp.float32)
    return jnp.maximum(acc * (1.0 / NN), 0.0) + res_ref[...]


def _mp_body(ng_ref, e_ref, res_ref, w2_ref, o_ref, t_ref):
    o_ref[...] = _mp_core(ng_ref, e_ref, res_ref, w2_ref, t_ref)


def _mp_fc_body(ng_ref, e_ref, res_ref, w2_ref, fw0_ref, fb0_ref, fw1_ref,
                fb1_ref, o_ref, t_ref):
    nodes2 = _mp_core(ng_ref, e_ref, res_ref, w2_ref, t_ref)
    h = jnp.maximum(nodes2 @ fw0_ref[...] + fb0_ref[...], 0.0)
    o_ref[...] = jnp.tanh(h @ fw1_ref[...] + fb1_ref[...])


_MP_SPECS = [
    pl.BlockSpec((_BN * NN, 128), lambda i: (i, 0)),
    pl.BlockSpec((_BN * NN * EF // 128, 128), lambda i: (i, 0)),
    pl.BlockSpec((_BN, 128), lambda i: (i, 0)),
    pl.BlockSpec((16, 128, 128), lambda i: (0, 0, 0)),
]
_FC_SPECS = [
    pl.BlockSpec((128, 128), lambda i: (0, 0)),
    pl.BlockSpec((1, 128), lambda i: (0, 0)),
    pl.BlockSpec((128, 128), lambda i: (0, 0)),
    pl.BlockSpec((1, 128), lambda i: (0, 0)),
]


def _mp(ng, e, res, w2):
    return pl.pallas_call(
        _mp_body,
        grid=(_GRID,),
        in_specs=_MP_SPECS,
        out_specs=pl.BlockSpec((_BN, 128), lambda i: (i, 0)),
        out_shape=jax.ShapeDtypeStruct((N, 128), jnp.float32),
        scratch_shapes=[pltpu.VMEM((16, _BN, 128), jnp.bfloat16)],
    )(ng, e, res, w2)


def _mp_fc(ng, e, res, w2, fw0, fb0, fw1, fb1):
    return pl.pallas_call(
        _mp_fc_body,
        grid=(_GRID,),
        in_specs=_MP_SPECS + _FC_SPECS,
        out_specs=pl.BlockSpec((_BN, 128), lambda i: (i, 0)),
        out_shape=jax.ShapeDtypeStruct((N, 128), jnp.float32),
        scratch_shapes=[pltpu.VMEM((16, _BN, 128), jnp.bfloat16)],
    )(ng, e, res, w2, fw0, fb0, fw1, fb1)


def kernel(node_input, nlist_input, edge_input, edge_w0, edge_b0, edge_w1,
           edge_b1, mp_w0, mp_w1, fc_w0, fc_b0, fc_w1, fc_b1):
    f32 = jnp.float32
    eye8 = jnp.eye(8, dtype=f32)
    w0b = jnp.kron(eye8, edge_w0.astype(f32))
    w1b = jnp.kron(eye8, edge_w1.astype(f32))
    b0b = jnp.tile(edge_b0.astype(f32), 8).reshape(1, 128)
    b1b = jnp.tile(edge_b1.astype(f32), 8).reshape(1, 128)

    x2 = (edge_input.astype(f32)
          .reshape(_GRID, 8, _BN * NN // 8, EF)
          .transpose(0, 2, 1, 3)
          .reshape(N * NN * EF // 128, 128))
    edges_packed = _edge_fc(x2, w0b, b0b, w1b, b1b)

    idx_flat = nlist_input.astype(jnp.int32).reshape(_NW, _RPW)
    idx = jnp.pad(idx_flat, ((0, 0), (0, _IDXROWS * _CH - _RPW))
                  ).reshape(_NW, _IDXROWS, _CH)
    w2a = jnp.transpose(mp_w0.astype(jnp.bfloat16), (1, 0, 2))
    w2b = jnp.transpose(mp_w1.astype(jnp.bfloat16), (1, 0, 2))

    node_f32 = node_input.astype(f32)
    ng1 = _gather(node_f32, idx)
    nodes1 = _mp(ng1, edges_packed, node_f32, w2a)
    ng2 = _gather(nodes1, idx)
    return _mp_fc(ng2, edges_packed, nodes1, w2b,
                  fc_w0.astype(f32), fc_b0.astype(f32).reshape(1, 128),
                  fc_w1.astype(f32), fc_b1.astype(f32).reshape(1, 128))


# speedup vs baseline: 1.1648x; 1.0712x over previous
import functools

import jax
import jax.numpy as jnp
from jax import lax
from jax.experimental import pallas as pl
from jax.experimental.pallas import tpu as pltpu
from jax.experimental.pallas import tpu_sc as plsc

N = 10000
NN = 16
F = 128
EF = 16

_NC = 2
_NS = 16
_NW = _NC * _NS


_BN = 400
_GRID = N // _BN


def _edge_fc_body(x_ref, w0_ref, b0_ref, w1_ref, b1_ref, o_ref):
    x = x_ref[...]
    h = jnp.maximum(x @ w0_ref[...] + b0_ref[...], 0.0)
    o_ref[...] = jnp.tanh(h @ w1_ref[...] + b1_ref[...])


def _edge_fc(x2, w0b, b0b, w1b, b1b):
    rows = x2.shape[0]
    blk = rows // 4
    return pl.pallas_call(
        _edge_fc_body,
        grid=(4,),
        in_specs=[
            pl.BlockSpec((blk, 128), lambda i: (i, 0)),
            pl.BlockSpec((128, 128), lambda i: (0, 0)),
            pl.BlockSpec((1, 128), lambda i: (0, 0)),
            pl.BlockSpec((128, 128), lambda i: (0, 0)),
            pl.BlockSpec((1, 128), lambda i: (0, 0)),
        ],
        out_specs=pl.BlockSpec((blk, 128), lambda i: (i, 0)),
        out_shape=jax.ShapeDtypeStruct((rows, 128), jnp.float32),
    )(x2, w0b, b0b, w1b, b1b)


_RPW = (N * NN) // _NW
_CH = 120
_NFULL = _RPW // _CH
_TAIL = _RPW - _NFULL * _CH
_IDXROWS = _NFULL + 1


def _gather_body(table_hbm, idx_hbm, out_hbm, spmem, idx_v, buf0, buf1,
                 sem0, sem1, sem_s):
    c = lax.axis_index("c")
    s = lax.axis_index("s")
    wid = s * _NC + c

    @pl.when(s == 0)
    def _stage():
        pltpu.async_copy(table_hbm, spmem, sem_s).wait()

    plsc.subcore_barrier()
    pltpu.sync_copy(idx_hbm.at[wid], idx_v)
    base = wid * _RPW

    def _gath(g, buf, sem):
        return pltpu.async_copy(spmem.at[idx_v.at[g]], buf, sem)

    def _scat(g, buf):
        pltpu.sync_copy(buf, out_hbm.at[pl.ds(base + g * _CH, _CH)])

    _gath(0, buf0, sem0)

    def body(it, carry):
        a = 2 * it
        pltpu.make_async_copy(spmem.at[idx_v.at[a]], buf0, sem0).wait()
        _gath(a + 1, buf1, sem1)
        _scat(a, buf0)
        pltpu.make_async_copy(spmem.at[idx_v.at[a + 1]], buf1, sem1).wait()
        _gath(a + 2, buf0, sem0)
        _scat(a + 1, buf1)
        return carry

    lax.fori_loop(0, _NFULL // 2, body, 0)
    last = _NFULL - 1
    pltpu.make_async_copy(spmem.at[idx_v.at[last]], buf0, sem0).wait()
    pltpu.async_copy(spmem.at[idx_v.at[_NFULL, pl.ds(0, _TAIL)]],
                     buf1.at[pl.ds(0, _TAIL)], sem1)
    _scat(last, buf0)
    pltpu.make_async_copy(spmem.at[idx_v.at[_NFULL, pl.ds(0, _TAIL)]],
                          buf1.at[pl.ds(0, _TAIL)], sem1).wait()
    pltpu.sync_copy(buf1.at[pl.ds(0, _TAIL)],
                    out_hbm.at[pl.ds(base + _NFULL * _CH, _TAIL)])


_gather = functools.partial(
    pl.kernel,
    out_type=jax.ShapeDtypeStruct((N * NN, F), jnp.float32),
    mesh=plsc.VectorSubcoreMesh(core_axis_name="c", subcore_axis_name="s"),
    scratch_types=[
        pltpu.VMEM_SHARED((N, F), jnp.float32),
        pltpu.VMEM((_IDXROWS, _CH), jnp.int32),
        pltpu.VMEM((_CH, F), jnp.float32),
        pltpu.VMEM((_CH, F), jnp.float32),
        pltpu.SemaphoreType.DMA,
        pltpu.SemaphoreType.DMA,
        pltpu.SemaphoreType.DMA,
    ],
)(_gather_body)


def _mp_core(ng_ref, e_ref, res_ref, w2_ref, t_ref):
    B = res_ref.shape[0]
    G = B // 16
    bf16 = jnp.bfloat16
    ep = e_ref[...]
    ng = ng_ref[...].astype(bf16)

    ci = lax.broadcasted_iota(jnp.int32, (16, 256), 1)
    ri = lax.broadcasted_iota(jnp.int32, (16, 256), 0)
    spread = (ci // 16 == ri).astype(bf16)
    r2 = lax.broadcasted_iota(jnp.int32, (256, 256), 0)
    c2 = lax.broadcasted_iota(jnp.int32, (256, 256), 1)
    mask = ((r2 % 32) // 2 == (c2 % 16)).astype(bf16)
    for g in range(G):
        eg = jnp.concatenate(
            [ep[32 * g:32 * (g + 1), EF * a:EF * (a + 1)] for a in range(8)],
            axis=0).astype(bf16)
        ngg = ng[g * 256:(g + 1) * 256, :]
        erep = lax.dot_general(eg, spread, (((1,), (0,)), ((), ())),
                               preferred_element_type=jnp.float32)
        ablk = erep.astype(bf16) * mask
        rg = lax.dot_general(ablk, ngg, (((0,), (0,)), ((), ())),
                             preferred_element_type=jnp.float32)
        for n in range(16):
            t_ref[n, g * 16:(g + 1) * 16, :] = rg[n * 16:(n + 1) * 16, :].astype(bf16)
    acc = jnp.zeros((B, 128), jnp.float32)
    for n in range(16):
        acc = acc + lax.dot_general(t_ref[n], w2_ref[n],
                                    (((1,), (0,)), ((), ())),
                                    preferred_element_type=jnp.float32)
    return jnp.maximum(acc * (1.0 / NN), 0.0) + res_ref[...]


def _mp_body(ng_ref, e_ref, res_ref, w2_ref, o_ref, t_ref):
    o_ref[...] = _mp_core(ng_ref, e_ref, res_ref, w2_ref, t_ref)


def _mp_fc_body(ng_ref, e_ref, res_ref, w2_ref, fw0_ref, fb0_ref, fw1_ref,
                fb1_ref, o_ref, t_ref):
    nodes2 = _mp_core(ng_ref, e_ref, res_ref, w2_ref, t_ref)
    h = jnp.maximum(nodes2 @ fw0_ref[...] + fb0_ref[...], 0.0)
    o_ref[...] = jnp.tanh(h @ fw1_ref[...] + fb1_ref[...])


_MP_SPECS = [
    pl.BlockSpec((_BN * NN, 128), lambda i: (i, 0)),
    pl.BlockSpec((_BN * NN * EF // 128, 128), lambda i: (i, 0)),
    pl.BlockSpec((_BN, 128), lambda i: (i, 0)),
    pl.BlockSpec((16, 128, 128), lambda i: (0, 0, 0)),
]
_FC_SPECS = [
    pl.BlockSpec((128, 128), lambda i: (0, 0)),
    pl.BlockSpec((1, 128), lambda i: (0, 0)),
    pl.BlockSpec((128, 128), lambda i: (0, 0)),
    pl.BlockSpec((1, 128), lambda i: (0, 0)),
]


def _mp(ng, e, res, w2):
    return pl.pallas_call(
        _mp_body,
        grid=(_GRID,),
        in_specs=_MP_SPECS,
        out_specs=pl.BlockSpec((_BN, 128), lambda i: (i, 0)),
        out_shape=jax.ShapeDtypeStruct((N, 128), jnp.float32),
        scratch_shapes=[pltpu.VMEM((16, _BN, 128), jnp.bfloat16)],
    )(ng, e, res, w2)


def _mp_fc(ng, e, res, w2, fw0, fb0, fw1, fb1):
    return pl.pallas_call(
        _mp_fc_body,
        grid=(_GRID,),
        in_specs=_MP_SPECS + _FC_SPECS,
        out_specs=pl.BlockSpec((_BN, 128), lambda i: (i, 0)),
        out_shape=jax.ShapeDtypeStruct((N, 128), jnp.float32),
        scratch_shapes=[pltpu.VMEM((16, _BN, 128), jnp.bfloat16)],
    )(ng, e, res, w2, fw0, fb0, fw1, fb1)


def kernel(node_input, nlist_input, edge_input, edge_w0, edge_b0, edge_w1,
           edge_b1, mp_w0, mp_w1, fc_w0, fc_b0, fc_w1, fc_b1):
    f32 = jnp.float32
    eye8 = jnp.eye(8, dtype=f32)
    w0b = jnp.kron(eye8, edge_w0.astype(f32))
    w1b = jnp.kron(eye8, edge_w1.astype(f32))
    b0b = jnp.tile(edge_b0.astype(f32), 8).reshape(1, 128)
    b1b = jnp.tile(edge_b1.astype(f32), 8).reshape(1, 128)

    x2 = edge_input.astype(f32).reshape(N * NN * EF // 128, 128)
    edges_packed = _edge_fc(x2, w0b, b0b, w1b, b1b)

    idx_flat = (nlist_input.astype(jnp.int32)
                .reshape(N * NN // 256, 32, 8).transpose(0, 2, 1)
                .reshape(_NW, _RPW))
    idx = jnp.pad(idx_flat, ((0, 0), (0, _IDXROWS * _CH - _RPW))
                  ).reshape(_NW, _IDXROWS, _CH)
    w2a = jnp.transpose(mp_w0.astype(jnp.bfloat16), (1, 0, 2))
    w2b = jnp.transpose(mp_w1.astype(jnp.bfloat16), (1, 0, 2))

    node_f32 = node_input.astype(f32)
    ng1 = _gather(node_f32, idx)
    nodes1 = _mp(ng1, edges_packed, node_f32, w2a)
    ng2 = _gather(nodes1, idx)
    return _mp_fc(ng2, edges_packed, nodes1, w2b,
                  fc_w0.astype(f32), fc_b0.astype(f32).reshape(1, 128),
                  fc_w1.astype(f32), fc_b1.astype(f32).reshape(1, 128))


# speedup vs baseline: 1.2381x; 1.0629x over previous
import functools

import jax
import jax.numpy as jnp
from jax import lax
from jax.experimental import pallas as pl
from jax.experimental.pallas import tpu as pltpu
from jax.experimental.pallas import tpu_sc as plsc

N = 10000
NN = 16
F = 128
EF = 16

_NC = 2
_NS = 16
_NW = _NC * _NS


_BN = 400
_GRID = N // _BN


def _edge_fc_body(x_ref, w0_ref, b0_ref, w1_ref, b1_ref, o_ref):
    x = x_ref[...]
    h = jnp.maximum(x @ w0_ref[...] + b0_ref[...], 0.0)
    o_ref[...] = jnp.tanh(h @ w1_ref[...] + b1_ref[...])


def _edge_fc(x2, w0b, b0b, w1b, b1b):
    rows = x2.shape[0]
    blk = rows // 4
    return pl.pallas_call(
        _edge_fc_body,
        grid=(4,),
        in_specs=[
            pl.BlockSpec((blk, 128), lambda i: (i, 0)),
            pl.BlockSpec((128, 128), lambda i: (0, 0)),
            pl.BlockSpec((1, 128), lambda i: (0, 0)),
            pl.BlockSpec((128, 128), lambda i: (0, 0)),
            pl.BlockSpec((1, 128), lambda i: (0, 0)),
        ],
        out_specs=pl.BlockSpec((blk, 128), lambda i: (i, 0)),
        out_shape=jax.ShapeDtypeStruct((rows, 128), jnp.float32),
    )(x2, w0b, b0b, w1b, b1b)


_RPW = (N * NN) // _NW
_CH = 120
_NFULL = _RPW // _CH
_TAIL = _RPW - _NFULL * _CH
_IDXROWS = _NFULL + 1


def _gather_body(table_hbm, idx_hbm, out_hbm, spmem, idx_v, buf0, buf1,
                 sem0, sem1, sem_s):
    c = lax.axis_index("c")
    s = lax.axis_index("s")
    wid = s * _NC + c

    @pl.when(s == 0)
    def _stage():
        pltpu.async_copy(table_hbm, spmem, sem_s).wait()

    plsc.subcore_barrier()
    pltpu.sync_copy(idx_hbm.at[wid], idx_v)
    base = wid * _RPW

    def _gath(g, buf, sem):
        return pltpu.async_copy(spmem.at[idx_v.at[g]], buf, sem)

    def _scat(g, buf):
        pltpu.sync_copy(buf, out_hbm.at[pl.ds(base + g * _CH, _CH)])

    _gath(0, buf0, sem0)

    def body(it, carry):
        a = 2 * it
        pltpu.make_async_copy(spmem.at[idx_v.at[a]], buf0, sem0).wait()
        _gath(a + 1, buf1, sem1)
        _scat(a, buf0)
        pltpu.make_async_copy(spmem.at[idx_v.at[a + 1]], buf1, sem1).wait()
        _gath(a + 2, buf0, sem0)
        _scat(a + 1, buf1)
        return carry

    lax.fori_loop(0, _NFULL // 2, body, 0)
    last = _NFULL - 1
    pltpu.make_async_copy(spmem.at[idx_v.at[last]], buf0, sem0).wait()
    pltpu.async_copy(spmem.at[idx_v.at[_NFULL, pl.ds(0, _TAIL)]],
                     buf1.at[pl.ds(0, _TAIL)], sem1)
    _scat(last, buf0)
    pltpu.make_async_copy(spmem.at[idx_v.at[_NFULL, pl.ds(0, _TAIL)]],
                          buf1.at[pl.ds(0, _TAIL)], sem1).wait()
    pltpu.sync_copy(buf1.at[pl.ds(0, _TAIL)],
                    out_hbm.at[pl.ds(base + _NFULL * _CH, _TAIL)])


_gather = functools.partial(
    pl.kernel,
    out_type=jax.ShapeDtypeStruct((N * NN, F), jnp.float32),
    mesh=plsc.VectorSubcoreMesh(core_axis_name="c", subcore_axis_name="s"),
    scratch_types=[
        pltpu.VMEM_SHARED((N, F), jnp.float32),
        pltpu.VMEM((_IDXROWS, _CH), jnp.int32),
        pltpu.VMEM((_CH, F), jnp.float32),
        pltpu.VMEM((_CH, F), jnp.float32),
        pltpu.SemaphoreType.DMA,
        pltpu.SemaphoreType.DMA,
        pltpu.SemaphoreType.DMA,
    ],
)(_gather_body)


def _mp_core(ng_ref, e_ref, res_ref, w2_ref, t_ref):
    B = res_ref.shape[0]
    G = B // 16
    bf16 = jnp.bfloat16
    ep = e_ref[...]
    ng = ng_ref[...].astype(bf16)

    ci = lax.broadcasted_iota(jnp.int32, (16, 256), 1)
    ri = lax.broadcasted_iota(jnp.int32, (16, 256), 0)
    spread = (ci // 16 == ri).astype(bf16)
    r2 = lax.broadcasted_iota(jnp.int32, (256, 256), 0)
    c2 = lax.broadcasted_iota(jnp.int32, (256, 256), 1)
    mask = ((r2 % 32) // 2 == (c2 % 16)).astype(bf16)
    for g in range(G):
        eg = jnp.concatenate(
            [ep[32 * g:32 * (g + 1), EF * a:EF * (a + 1)] for a in range(8)],
            axis=0).astype(bf16)
        ngg = ng[g * 256:(g + 1) * 256, :]
        erep = lax.dot_general(eg, spread, (((1,), (0,)), ((), ())),
                               preferred_element_type=jnp.float32)
        ablk = erep.astype(bf16) * mask
        rg = lax.dot_general(ablk, ngg, (((0,), (0,)), ((), ())),
                             preferred_element_type=jnp.float32)
        for n in range(16):
            t_ref[g * 16:(g + 1) * 16, n * 128:(n + 1) * 128] = (
                rg[n * 16:(n + 1) * 16, :].astype(bf16))
    acc = lax.dot_general(t_ref[...], w2_ref[...],
                          (((1,), (0,)), ((), ())),
                          preferred_element_type=jnp.float32)
    return jnp.maximum(acc * (1.0 / NN), 0.0) + res_ref[...]


def _mp_body(ng_ref, e_ref, res_ref, w2_ref, o_ref, t_ref):
    o_ref[...] = _mp_core(ng_ref, e_ref, res_ref, w2_ref, t_ref)


def _mp_fc_body(ng_ref, e_ref, res_ref, w2_ref, fw0_ref, fb0_ref, fw1_ref,
                fb1_ref, o_ref, t_ref):
    nodes2 = _mp_core(ng_ref, e_ref, res_ref, w2_ref, t_ref)
    h = jnp.maximum(nodes2 @ fw0_ref[...] + fb0_ref[...], 0.0)
    o_ref[...] = jnp.tanh(h @ fw1_ref[...] + fb1_ref[...])


_MP_SPECS = [
    pl.BlockSpec((_BN * NN, 128), lambda i: (i, 0)),
    pl.BlockSpec((_BN * NN * EF // 128, 128), lambda i: (i, 0)),
    pl.BlockSpec((_BN, 128), lambda i: (i, 0)),
    pl.BlockSpec((NN * 128, 128), lambda i: (0, 0)),
]
_FC_SPECS = [
    pl.BlockSpec((128, 128), lambda i: (0, 0)),
    pl.BlockSpec((1, 128), lambda i: (0, 0)),
    pl.BlockSpec((128, 128), lambda i: (0, 0)),
    pl.BlockSpec((1, 128), lambda i: (0, 0)),
]


def _mp(ng, e, res, w2):
    return pl.pallas_call(
        _mp_body,
        grid=(_GRID,),
        in_specs=_MP_SPECS,
        out_specs=pl.BlockSpec((_BN, 128), lambda i: (i, 0)),
        out_shape=jax.ShapeDtypeStruct((N, 128), jnp.float32),
        scratch_shapes=[pltpu.VMEM((_BN, NN * 128), jnp.bfloat16)],
    )(ng, e, res, w2)


def _mp_fc(ng, e, res, w2, fw0, fb0, fw1, fb1):
    return pl.pallas_call(
        _mp_fc_body,
        grid=(_GRID,),
        in_specs=_MP_SPECS + _FC_SPECS,
        out_specs=pl.BlockSpec((_BN, 128), lambda i: (i, 0)),
        out_shape=jax.ShapeDtypeStruct((N, 128), jnp.float32),
        scratch_shapes=[pltpu.VMEM((_BN, NN * 128), jnp.bfloat16)],
    )(ng, e, res, w2, fw0, fb0, fw1, fb1)


def kernel(node_input, nlist_input, edge_input, edge_w0, edge_b0, edge_w1,
           edge_b1, mp_w0, mp_w1, fc_w0, fc_b0, fc_w1, fc_b1):
    f32 = jnp.float32
    eye8 = jnp.eye(8, dtype=f32)
    w0b = jnp.kron(eye8, edge_w0.astype(f32))
    w1b = jnp.kron(eye8, edge_w1.astype(f32))
    b0b = jnp.tile(edge_b0.astype(f32), 8).reshape(1, 128)
    b1b = jnp.tile(edge_b1.astype(f32), 8).reshape(1, 128)

    x2 = edge_input.astype(f32).reshape(N * NN * EF // 128, 128)
    edges_packed = _edge_fc(x2, w0b, b0b, w1b, b1b)

    idx_flat = (nlist_input.astype(jnp.int32)
                .reshape(N * NN // 256, 32, 8).transpose(0, 2, 1)
                .reshape(_NW, _RPW))
    idx = jnp.pad(idx_flat, ((0, 0), (0, _IDXROWS * _CH - _RPW))
                  ).reshape(_NW, _IDXROWS, _CH)
    w2a = jnp.transpose(mp_w0.astype(jnp.bfloat16), (1, 0, 2)).reshape(EF * F, F)
    w2b = jnp.transpose(mp_w1.astype(jnp.bfloat16), (1, 0, 2)).reshape(EF * F, F)

    node_f32 = node_input.astype(f32)
    ng1 = _gather(node_f32, idx)
    nodes1 = _mp(ng1, edges_packed, node_f32, w2a)
    ng2 = _gather(nodes1, idx)
    return _mp_fc(ng2, edges_packed, nodes1, w2b,
                  fc_w0.astype(f32), fc_b0.astype(f32).reshape(1, 128),
                  fc_w1.astype(f32), fc_b1.astype(f32).reshape(1, 128))
